# baseline (speedup 1.0000x reference)
import functools
import jax
import jax.numpy as jnp
from jax.experimental import pallas as pl

_N = 10000
_E = 320000
_D_IN = 128
_OUT = 128
_HEADS = 8
_ETYPES = 16

_NP = 10240
_BE = 8192
_EP = 327680
_BM = 1024


def _dense_body(x_ref, w_ref, as_ref, ad_ref, b_ref, h_ref, asrc_ref, adst_ref,
                *, heads, out_c, relu_in):
    x = x_ref[...]
    if relu_in:
        x = jnp.maximum(x + b_ref[...][0][None, :], 0.0)
    h = jnp.dot(x, w_ref[...], preferred_element_type=jnp.float32)
    h_ref[...] = h
    hr = h.reshape(x.shape[0], heads, out_c)
    asrc_ref[...] = (hr * as_ref[...][None]).sum(-1)
    adst_ref[...] = (hr * ad_ref[...][None]).sum(-1)


def _dense(x, w, att_src, att_dst, b, heads, out_c, relu_in, bn=1024):
    n = x.shape[0]
    d = x.shape[1]
    body = functools.partial(_dense_body, heads=heads, out_c=out_c,
                             relu_in=relu_in)
    return pl.pallas_call(
        body,
        grid=(n // bn,),
        in_specs=[
            pl.BlockSpec((bn, d), lambda i: (i, 0)),
            pl.BlockSpec((d, heads * out_c), lambda i: (0, 0)),
            pl.BlockSpec((heads, out_c), lambda i: (0, 0)),
            pl.BlockSpec((heads, out_c), lambda i: (0, 0)),
            pl.BlockSpec((1, d), lambda i: (0, 0)),
        ],
        out_specs=[
            pl.BlockSpec((bn, heads * out_c), lambda i: (i, 0)),
            pl.BlockSpec((bn, heads), lambda i: (i, 0)),
            pl.BlockSpec((bn, heads), lambda i: (i, 0)),
        ],
        out_shape=[
            jax.ShapeDtypeStruct((n, heads * out_c), jnp.float32),
            jax.ShapeDtypeStruct((n, heads), jnp.float32),
            jax.ShapeDtypeStruct((n, heads), jnp.float32),
        ],
    )(x, w, att_src, att_dst, b.reshape(1, d))


def _etab_body(emb_ref, we_ref, ae_ref, out_ref, *, heads, out_c):
    et = jnp.dot(emb_ref[...], we_ref[...], preferred_element_type=jnp.float32)
    etr = et.reshape(_ETYPES, heads, out_c)
    out_ref[...] = (etr * ae_ref[...][None]).sum(-1)


def _etab(emb, we, att_e, heads, out_c):
    body = functools.partial(_etab_body, heads=heads, out_c=out_c)
    return pl.pallas_call(
        body,
        out_shape=jax.ShapeDtypeStruct((_ETYPES, heads), jnp.float32),
    )(emb, we, att_e)


def _alpha_body(a_ref, b_ref, c_ref, o_ref):
    s = a_ref[...] + b_ref[...] + c_ref[...]
    o_ref[...] = jnp.where(s > 0, s, 0.2 * s)


def _ex_body(al_ref, mx_ref, o_ref):
    o_ref[...] = jnp.exp(al_ref[...] - mx_ref[...])


def _edge_ew(body, args, heads):
    return pl.pallas_call(
        body,
        grid=(_EP // _BE,),
        in_specs=[pl.BlockSpec((_BE, heads), lambda i: (i, 0))] * len(args),
        out_specs=pl.BlockSpec((_BE, heads), lambda i: (i, 0)),
        out_shape=jax.ShapeDtypeStruct((_EP, heads), jnp.float32),
    )(*args)


def _msg_body(ex_ref, den_ref, h_ref, o_ref, *, heads, out_c):
    att = ex_ref[...] / (den_ref[...] + 1e-16)
    o_ref[...] = h_ref[...] * att[:, :, None]


def _msg(ex, den, h_src, heads, out_c):
    body = functools.partial(_msg_body, heads=heads, out_c=out_c)
    return pl.pallas_call(
        body,
        grid=(_EP // _BM,),
        in_specs=[
            pl.BlockSpec((_BM, heads), lambda i: (i, 0)),
            pl.BlockSpec((_BM, heads), lambda i: (i, 0)),
            pl.BlockSpec((_BM, heads, out_c), lambda i: (i, 0, 0)),
        ],
        out_specs=pl.BlockSpec((_BM, heads, out_c), lambda i: (i, 0, 0)),
        out_shape=jax.ShapeDtypeStruct((_EP, heads, out_c), jnp.float32),
    )(ex, den, h_src)


def _gat_layer(h, asrc, adst, ae_tab, src, dst, etype, heads, out_c):
    ae_e = ae_tab[etype]
    asrc_e = asrc[src]
    adst_e = adst[dst]
    alpha = _edge_ew(_alpha_body, (asrc_e, adst_e, ae_e), heads)
    amax = jax.ops.segment_max(alpha, dst, num_segments=_NP,
                               indices_are_sorted=True)
    amax = jnp.where(jnp.isfinite(amax), amax, 0.0)
    ex = _edge_ew(_ex_body, (alpha, amax[dst]), heads)
    den = jax.ops.segment_sum(ex, dst, num_segments=_NP,
                              indices_are_sorted=True)
    hr = h.reshape(_NP, heads, out_c)
    msg = _msg(ex, den[dst], hr[src], heads, out_c)
    out = jax.ops.segment_sum(msg, dst, num_segments=_NP,
                              indices_are_sorted=True)
    return out.reshape(_NP, heads * out_c)


def kernel(x, edge_index, edge_attr, emb_table, W1, att_src1, att_dst1, We1,
           att_e1, b1, W2, att_src2, att_dst2, We2, att_e2, b2):
    src = edge_index[0]
    dst = edge_index[1]
    perm = jnp.argsort(dst)
    src = src[perm]
    dst = dst[perm]
    etype = edge_attr[perm]

    src = jnp.concatenate([src, jnp.zeros((_EP - _E,), jnp.int32)])
    dst = jnp.concatenate([dst, jnp.full((_EP - _E,), _NP - 1, jnp.int32)])
    etype = jnp.concatenate([etype, jnp.zeros((_EP - _E,), jnp.int32)])

    xp = jnp.concatenate([x, jnp.zeros((_NP - _N, _D_IN), jnp.float32)])

    h1, asrc1, adst1 = _dense(xp, W1, att_src1, att_dst1,
                              jnp.zeros((_D_IN,), jnp.float32),
                              _HEADS, _OUT, relu_in=False)
    ae_tab1 = _etab(emb_table, We1, att_e1, _HEADS, _OUT)
    out1 = _gat_layer(h1, asrc1, adst1, ae_tab1, src, dst, etype,
                      _HEADS, _OUT)

    h2, asrc2, adst2 = _dense(out1, W2, att_src2, att_dst2, b1,
                              1, _OUT, relu_in=True)
    ae_tab2 = _etab(emb_table, We2, att_e2, 1, _OUT)
    out2 = _gat_layer(h2, asrc2, adst2, ae_tab2, src, dst, etype, 1, _OUT)

    return out2[:_N] + b2
